# Initial kernel scaffold; baseline (speedup 1.0000x reference)
import jax
import jax.numpy as jnp
from jax.experimental import pallas as pl


def kernel(x, gate_w, gate_b, w1, w2):
    raise NotImplementedError("write your pallas kernel here")


# speedup vs baseline: 1.6521x; 1.6521x over previous
import functools

import jax
import jax.numpy as jnp
from jax.experimental import pallas as pl
from jax.experimental.pallas import tpu as pltpu

NUM_EXPERTS = 8
TOP_K = 2
D_MODEL = 1024
D_HID = 2048
AUX_COEFF = 0.01

T_TOKENS = 2048
BH = 512
NHB = D_HID // BH


def _router_body(x_ref, gwt_ref, gb_ref, comb_ref, aux_ref):
    T, E = T_TOKENS, NUM_EXPERTS
    logits = jnp.dot(x_ref[...], gwt_ref[...],
                     preferred_element_type=jnp.float32) + gb_ref[...]
    ids = jax.lax.broadcasted_iota(jnp.int32, (T, E), 1)
    m1 = jnp.max(logits, axis=1, keepdims=True)
    i1 = jnp.min(jnp.where(logits == m1, ids, E), axis=1, keepdims=True)
    neg = jnp.float32(-jnp.inf)
    logits_m = jnp.where(ids == i1, neg, logits)
    m2 = jnp.max(logits_m, axis=1, keepdims=True)
    i2 = jnp.min(jnp.where(logits_m == m2, ids, E), axis=1, keepdims=True)
    e21 = jnp.exp(m2 - m1)
    g1 = 1.0 / (1.0 + e21)
    g2 = e21 / (1.0 + e21)
    comb_ref[...] = (jnp.where(ids == i1, g1, 0.0)
                     + jnp.where(ids == i2, g2, 0.0))
    lse = m1 + jnp.log(jnp.sum(jnp.exp(logits - m1), axis=1, keepdims=True))
    aux = (AUX_COEFF / E) * (-jnp.log(jnp.float32(E))
                             - jnp.mean(logits) + jnp.mean(lse))
    aux_ref[...] = jnp.reshape(aux, (1, 1))


def _router(x2d, gate_w, gate_b, interpret=False):
    T, E = T_TOKENS, NUM_EXPERTS
    comb, aux = pl.pallas_call(
        _router_body,
        out_shape=(jax.ShapeDtypeStruct((T, E), jnp.float32),
                   jax.ShapeDtypeStruct((1, 1), jnp.float32)),
        interpret=interpret,
    )(x2d, gate_w.T, gate_b.reshape(1, E))
    return comb, aux[0, 0]


def _ffn_dense_body(x_ref, w1_ref, w2_ref, comb_ref, out_ref):
    e = pl.program_id(0)
    hb = pl.program_id(1)

    @pl.when(jnp.logical_and(e == 0, hb == 0))
    def _():
        out_ref[...] = jnp.zeros_like(out_ref)

    h = jax.nn.gelu(jnp.dot(x_ref[...], w1_ref[0],
                            preferred_element_type=jnp.float32))
    o = jnp.dot(h, w2_ref[0], preferred_element_type=jnp.float32)
    lane = jax.lax.broadcasted_iota(jnp.int32, (1, NUM_EXPERTS), 1)
    col = jnp.sum(comb_ref[...] * (lane == e).astype(jnp.float32),
                  axis=1, keepdims=True)
    out_ref[...] += col * o


def _ffn_dense(x2d, w1, w2, comb, interpret=False):
    T, D, H, E = T_TOKENS, D_MODEL, D_HID, NUM_EXPERTS
    return pl.pallas_call(
        _ffn_dense_body,
        grid=(E, NHB),
        in_specs=[
            pl.BlockSpec((T, D), lambda e, h: (0, 0)),
            pl.BlockSpec((1, D, BH), lambda e, h: (e, 0, h)),
            pl.BlockSpec((1, BH, D), lambda e, h: (e, h, 0)),
            pl.BlockSpec((T, E), lambda e, h: (0, 0)),
        ],
        out_specs=pl.BlockSpec((T, D), lambda e, h: (0, 0)),
        out_shape=jax.ShapeDtypeStruct((T, D), jnp.float32),
        interpret=interpret,
    )(x2d, w1, w2, comb)


def _moe(x, gate_w, gate_b, w1, w2, interpret=False):
    B, S, D = x.shape
    x2d = x.reshape(B * S, D)
    comb, aux = _router(x2d, gate_w, gate_b, interpret=interpret)
    out = _ffn_dense(x2d, w1, w2, comb, interpret=interpret)
    return out.reshape(B, S, D), aux


@jax.jit
def kernel(x, gate_w, gate_b, w1, w2):
    return _moe(x, gate_w, gate_b, w1, w2)
